# baseline (speedup 1.0000x reference)
import functools

import numpy as np

import jax
import jax.numpy as jnp
from jax import lax
from jax.experimental import pallas as pl
from jax.experimental.pallas import tpu as pltpu
from jax.experimental.pallas import tpu_sc as plsc

_N = 100000
_P = 64

_NC = 2
_NS = 16
_NW = _NC * _NS


def _perm(vec, pattern):
    return vec.at[pattern].get(mode="promise_in_bounds")


def _make_sc_kernel(n_rows, rows_chunk, unroll):
    nominal = n_rows // _NW
    assert n_rows % 8 == 0 and rows_chunk % 8 == 0
    sizes = [
        (nominal * (w + 1)) // 8 * 8 - (nominal * w) // 8 * 8 for w in range(_NW)
    ]
    n_chunks = -(-max(sizes) // rows_chunk)
    assert rows_chunk <= min(sizes)

    mesh = plsc.VectorSubcoreMesh(
        core_axis_name="c", subcore_axis_name="s",
        num_cores=_NC, num_subcores=_NS,
    )

    @functools.partial(
        pl.kernel,
        mesh=mesh,
        out_type=jax.ShapeDtypeStruct((n_rows, _P), jnp.float32),
        scratch_types=[
            pltpu.VMEM((rows_chunk, _P), jnp.float32),
            pltpu.VMEM((rows_chunk, _P), jnp.float32),
            pltpu.VMEM((rows_chunk, _P), jnp.float32),
            pltpu.VMEM((_P,), jnp.float32),
            pltpu.SemaphoreType.DMA,
            pltpu.SemaphoreType.DMA,
            pltpu.SemaphoreType.DMA,
            pltpu.SemaphoreType.DMA,
            pltpu.SemaphoreType.DMA,
            pltpu.SemaphoreType.DMA,
        ],
    )
    def knl(x_hbm, w_hbm, out_hbm, b0, b1, b2, w_v, i0, i1, i2, o0, o1, o2):
        bufs = [b0, b1, b2]
        isems = [i0, i1, i2]
        osems = [o0, o1, o2]
        wid = lax.axis_index("c") * _NS + lax.axis_index("s")
        start_w = (nominal * wid) // 8 * 8
        end_w = (nominal * (wid + 1)) // 8 * 8
        pltpu.sync_copy(w_hbm, w_v)
        wpos = [w_v[pl.ds(16 * k, 16)] for k in range(4)]
        lane = lax.iota(jnp.int32, 16)
        pat = {t: (lane + t) % 16 for t in (1, 5, 11, 15)}
        msk = {t: lane < 16 - t for t in (1, 5, 11, 15)}

        def window(vecs, i0, t):
            a = _perm(vecs[i0 % 4], pat[t])
            b = _perm(vecs[(i0 + 1) % 4], pat[t])
            return jnp.where(msk[t], a, b)

        def make_row_body(buf):
            def row_body(r):
                v = [buf[r, pl.ds(16 * k, 16)] for k in range(4)]
                ep = [jnp.exp(x) for x in v]
                en = [1.0 / e for e in ep]
                winv = []
                for k in range(4):
                    s = (
                        en[k]
                        + window(ep, k, 5)
                        + window(en, k, 11)
                        + window(ep, k + 1, 1)
                    )
                    winv.append(wpos[k] / s)
                for m in range(4):
                    a = window(winv, m + 3, 11) + window(winv, m + 2, 15)
                    b = winv[m] + window(winv, m + 3, 5)
                    buf[r, pl.ds(16 * m, 16)] = ep[m] * a - en[m] * b

            return row_body

        def r0_of(i):
            r = jnp.minimum(start_w + i * rows_chunk, end_w - rows_chunk)
            return pl.multiple_of(r, 8)

        in_cp = [None] * n_chunks
        out_cp = [None] * n_chunks
        in_cp[0] = pltpu.async_copy(
            x_hbm.at[pl.ds(r0_of(0), rows_chunk)], bufs[0], isems[0]
        )
        for i in range(n_chunks):
            b = i % 3
            if i + 1 < n_chunks:
                nb = (i + 1) % 3
                if i >= 2:
                    out_cp[i - 2].wait()
                in_cp[i + 1] = pltpu.async_copy(
                    x_hbm.at[pl.ds(r0_of(i + 1), rows_chunk)], bufs[nb], isems[nb]
                )
            in_cp[i].wait()
            plsc.parallel_loop(0, rows_chunk, unroll=unroll)(make_row_body(bufs[b]))
            out_cp[i] = pltpu.async_copy(
                bufs[b], out_hbm.at[pl.ds(r0_of(i), rows_chunk)], osems[b]
            )
        for i in range(max(0, n_chunks - 3), n_chunks):
            out_cp[i].wait()

    return knl


_sc_kernel = _make_sc_kernel(_N, 256, 5)


def kernel(inputs, clause_weights):
    return _sc_kernel(inputs, clause_weights)


# speedup vs baseline: 1.2120x; 1.2120x over previous
import functools

import numpy as np

import jax
import jax.numpy as jnp
from jax import lax
from jax.experimental import pallas as pl
from jax.experimental.pallas import tpu as pltpu
from jax.experimental.pallas import tpu_sc as plsc

_N = 100000
_P = 64

_NC = 2
_NS = 16
_NW = _NC * _NS


def _perm(vec, pattern):
    return vec.at[pattern].get(mode="promise_in_bounds")


def _make_sc_kernel(n_rows, rows_chunk, unroll):
    nominal = n_rows // _NW
    assert n_rows % 8 == 0 and rows_chunk % 8 == 0
    sizes = [
        (nominal * (w + 1)) // 8 * 8 - (nominal * w) // 8 * 8 for w in range(_NW)
    ]
    n_chunks = -(-max(sizes) // rows_chunk)
    assert rows_chunk <= min(sizes)

    mesh = plsc.VectorSubcoreMesh(
        core_axis_name="c", subcore_axis_name="s",
        num_cores=_NC, num_subcores=_NS,
    )

    @functools.partial(
        pl.kernel,
        mesh=mesh,
        out_type=jax.ShapeDtypeStruct((n_rows, _P), jnp.float32),
        scratch_types=[
            pltpu.VMEM((rows_chunk, _P), jnp.float32),
            pltpu.VMEM((rows_chunk, _P), jnp.float32),
            pltpu.VMEM((rows_chunk, _P), jnp.float32),
            pltpu.VMEM((_P,), jnp.float32),
            pltpu.SemaphoreType.DMA,
            pltpu.SemaphoreType.DMA,
            pltpu.SemaphoreType.DMA,
            pltpu.SemaphoreType.DMA,
            pltpu.SemaphoreType.DMA,
            pltpu.SemaphoreType.DMA,
        ],
    )
    def knl(x_hbm, w_hbm, out_hbm, b0, b1, b2, w_v, i0, i1, i2, o0, o1, o2):
        bufs = [b0, b1, b2]
        isems = [i0, i1, i2]
        osems = [o0, o1, o2]
        wid = lax.axis_index("c") * _NS + lax.axis_index("s")
        start_w = (nominal * wid) // 8 * 8
        end_w = (nominal * (wid + 1)) // 8 * 8
        pltpu.sync_copy(w_hbm, w_v)
        wpos = [w_v[pl.ds(16 * k, 16)] for k in range(4)]
        lane = lax.iota(jnp.int32, 16)
        pat = {t: (lane + t) % 16 for t in (1, 5, 11, 15)}
        msk = {t: lane < 16 - t for t in (1, 5, 11, 15)}

        def window(vecs, i0, t):
            a = _perm(vecs[i0 % 4], pat[t])
            b = _perm(vecs[(i0 + 1) % 4], pat[t])
            return jnp.where(msk[t], a, b)

        def make_row_body(buf):
            def row_body(r):
                v = [buf[r, pl.ds(16 * k, 16)] for k in range(4)]
                ep = [jnp.exp(x) for x in v]
                en = [1.0 / e for e in ep]
                winv = []
                for k in range(4):
                    s = (
                        en[k]
                        + window(ep, k, 5)
                        + window(en, k, 11)
                        + window(ep, k + 1, 1)
                    )
                    winv.append(wpos[k] / s)
                for m in range(4):
                    a = window(winv, m + 3, 11) + window(winv, m + 2, 15)
                    b = winv[m] + window(winv, m + 3, 5)
                    buf[r, pl.ds(16 * m, 16)] = ep[m] * a - en[m] * b

            return row_body

        def r0_of(i):
            r = jnp.minimum(start_w + i * rows_chunk, end_w - rows_chunk)
            return pl.multiple_of(r, 8)

        in_cp = [None] * n_chunks
        out_cp = [None] * n_chunks
        in_cp[0] = pltpu.async_copy(
            x_hbm.at[pl.ds(r0_of(0), rows_chunk)], bufs[0], isems[0]
        )
        for i in range(n_chunks):
            b = i % 3
            if i + 1 < n_chunks:
                nb = (i + 1) % 3
                if i >= 2:
                    out_cp[i - 2].wait()
                in_cp[i + 1] = pltpu.async_copy(
                    x_hbm.at[pl.ds(r0_of(i + 1), rows_chunk)], bufs[nb], isems[nb]
                )
            in_cp[i].wait()
            plsc.parallel_loop(0, rows_chunk, unroll=unroll)(make_row_body(bufs[b]))
            out_cp[i] = pltpu.async_copy(
                bufs[b], out_hbm.at[pl.ds(r0_of(i), rows_chunk)], osems[b]
            )
        for i in range(max(0, n_chunks - 3), n_chunks):
            out_cp[i].wait()

    return knl


_sc_kernel = _make_sc_kernel(_N, 288, 3)


def kernel(inputs, clause_weights):
    return _sc_kernel(inputs, clause_weights)
